# Initial kernel scaffold; baseline (speedup 1.0000x reference)
import jax
import jax.numpy as jnp
from jax.experimental import pallas as pl


def kernel(data, batch_id, batch_size, weights, bias):
    raise NotImplementedError("write your pallas kernel here")


# speedup vs baseline: 10.5387x; 10.5387x over previous
import functools

import jax
import jax.numpy as jnp
from jax import lax
from jax.experimental import pallas as pl
from jax.experimental.pallas import tpu as pltpu

IC = 128
NGROUP = 32
CPG = IC // NGROUP
EPSV = 1e-5
NSEG = 16


def _dot_t(a, b):
    return lax.dot_general(a, b, (((0,), (0,)), ((), ())),
                           preferred_element_type=jnp.float32)


def _onehot(bid_row, rows):
    seg = lax.broadcasted_iota(jnp.int32, (rows, NSEG), 1)
    return (bid_row[:, None] == seg).astype(jnp.float32)


def _p1_body(nblocks, x_ref, bid_ref, w_ref, b_ref, scale_ref, shift_ref,
             s1, s2, cnt):
    i = pl.program_id(0)

    @pl.when(i == 0)
    def _():
        s1[...] = jnp.zeros_like(s1)
        s2[...] = jnp.zeros_like(s2)
        cnt[...] = jnp.zeros_like(cnt)

    x = x_ref[...]
    rows = x.shape[0]
    oh = _onehot(bid_ref[0, 0, :], rows)
    s1[...] += _dot_t(oh, x)
    s2[...] += _dot_t(oh, x * x)
    cnt[...] += _dot_t(oh, jnp.ones_like(x))

    @pl.when(i == nblocks - 1)
    def _():
        ic = 1.0 / (cnt[...] * CPG + EPSV)
        ci = lax.broadcasted_iota(jnp.int32, (IC, IC), 0) // CPG
        cj = lax.broadcasted_iota(jnp.int32, (IC, IC), 1) // CPG
        ggt = (ci == cj).astype(jnp.float32)
        a1 = lax.dot_general(s1[...], ggt, (((1,), (0,)), ((), ())),
                             preferred_element_type=jnp.float32)
        a2 = lax.dot_general(s2[...], ggt, (((1,), (0,)), ((), ())),
                             preferred_element_type=jnp.float32)
        mg = a1 * ic
        var = ic * (a2 - 2.0 * mg * a1 + cnt[...] * CPG * mg * mg)
        istd = lax.rsqrt(var + EPSV)
        w = w_ref[...]
        scale_ref[...] = istd * w
        shift_ref[...] = b_ref[...] - mg * istd * w


def _p2_body(x_ref, bid_ref, scale_ref, shift_ref, o_ref):
    x = x_ref[...]
    oh = _onehot(bid_ref[0, 0, :], x.shape[0])
    rs = lax.dot_general(oh, scale_ref[...], (((1,), (0,)), ((), ())),
                         preferred_element_type=jnp.float32)
    rh = lax.dot_general(oh, shift_ref[...], (((1,), (0,)), ((), ())),
                         preferred_element_type=jnp.float32)
    o_ref[...] = x * rs + rh


def kernel(data, batch_id, batch_size, weights, bias):
    n, c = data.shape
    rows = 2000
    nblocks = n // rows
    assert nblocks * rows == n
    bid3 = batch_id.astype(jnp.int32).reshape(nblocks, 1, rows)

    scale, shift = pl.pallas_call(
        functools.partial(_p1_body, nblocks),
        grid=(nblocks,),
        in_specs=[
            pl.BlockSpec((rows, c), lambda i: (i, 0)),
            pl.BlockSpec((1, 1, rows), lambda i: (i, 0, 0)),
            pl.BlockSpec((1, c), lambda i: (0, 0)),
            pl.BlockSpec((1, c), lambda i: (0, 0)),
        ],
        out_specs=[
            pl.BlockSpec((NSEG, c), lambda i: (0, 0)),
            pl.BlockSpec((NSEG, c), lambda i: (0, 0)),
        ],
        out_shape=[
            jax.ShapeDtypeStruct((NSEG, c), jnp.float32),
            jax.ShapeDtypeStruct((NSEG, c), jnp.float32),
        ],
        scratch_shapes=[
            pltpu.VMEM((NSEG, c), jnp.float32),
            pltpu.VMEM((NSEG, c), jnp.float32),
            pltpu.VMEM((NSEG, c), jnp.float32),
        ],
        compiler_params=pltpu.CompilerParams(
            dimension_semantics=("arbitrary",)),
    )(data, bid3, weights, bias)

    out = pl.pallas_call(
        _p2_body,
        grid=(nblocks,),
        in_specs=[
            pl.BlockSpec((rows, c), lambda i: (i, 0)),
            pl.BlockSpec((1, 1, rows), lambda i: (i, 0, 0)),
            pl.BlockSpec((NSEG, c), lambda i: (0, 0)),
            pl.BlockSpec((NSEG, c), lambda i: (0, 0)),
        ],
        out_specs=pl.BlockSpec((rows, c), lambda i: (i, 0)),
        out_shape=jax.ShapeDtypeStruct((n, c), jnp.float32),
        compiler_params=pltpu.CompilerParams(
            dimension_semantics=("arbitrary",)),
    )(data, bid3, scale, shift)
    return out


# speedup vs baseline: 12.5180x; 1.1878x over previous
import functools

import jax
import jax.numpy as jnp
from jax import lax
from jax.experimental import pallas as pl
from jax.experimental.pallas import tpu as pltpu

IC = 128
NGROUP = 32
CPG = IC // NGROUP
EPSV = 1e-5
NSEG = 16


def _dot_t(a, b):
    return lax.dot_general(a, b, (((0,), (0,)), ((), ())),
                           preferred_element_type=jnp.float32)


def _onehot(bid_row, rows):
    seg = lax.broadcasted_iota(jnp.int32, (rows, NSEG), 1)
    return (bid_row[:, None] == seg).astype(jnp.float32)


def _body(nblocks, rows, x_ref, bid_ref, w_ref, b_ref, o_ref,
          xs, s1, s2, cnt):
    p = pl.program_id(0)
    j = pl.program_id(1)

    @pl.when((p == 0) & (j == 0))
    def _():
        s1[...] = jnp.zeros_like(s1)
        s2[...] = jnp.zeros_like(s2)
        cnt[...] = jnp.zeros_like(cnt)

    @pl.when(p == 0)
    def _():
        x = x_ref[...]
        xs[pl.ds(j * rows, rows), :] = x
        oh = _onehot(bid_ref[0, 0, :], rows)
        s1[...] += _dot_t(oh, x)
        s2[...] += _dot_t(oh, x * x)
        cnt[...] += _dot_t(oh, jnp.ones_like(x))

        @pl.when(j == nblocks - 1)
        def _():
            ic = 1.0 / (cnt[...] * CPG + EPSV)
            ci = lax.broadcasted_iota(jnp.int32, (IC, IC), 0) // CPG
            cj = lax.broadcasted_iota(jnp.int32, (IC, IC), 1) // CPG
            ggt = (ci == cj).astype(jnp.float32)
            a1 = lax.dot_general(s1[...], ggt, (((1,), (0,)), ((), ())),
                                 preferred_element_type=jnp.float32)
            a2 = lax.dot_general(s2[...], ggt, (((1,), (0,)), ((), ())),
                                 preferred_element_type=jnp.float32)
            mg = a1 * ic
            var = ic * (a2 - 2.0 * mg * a1 + cnt[...] * CPG * mg * mg)
            istd = lax.rsqrt(var + EPSV)
            w = w_ref[...]
            scale = istd * w
            shift = b_ref[...] - mg * scale
            s1[...] = scale
            s2[...] = shift

    @pl.when(p == 1)
    def _():
        x = xs[pl.ds(j * rows, rows), :]
        oh = _onehot(bid_ref[0, 0, :], rows)
        rs = lax.dot_general(oh, s1[...], (((1,), (0,)), ((), ())),
                             preferred_element_type=jnp.float32)
        rh = lax.dot_general(oh, s2[...], (((1,), (0,)), ((), ())),
                             preferred_element_type=jnp.float32)
        o_ref[...] = x * rs + rh


def kernel(data, batch_id, batch_size, weights, bias):
    n, c = data.shape
    rows = 2000
    nblocks = n // rows
    assert nblocks * rows == n
    bid3 = batch_id.astype(jnp.int32).reshape(nblocks, 1, rows)

    out = pl.pallas_call(
        functools.partial(_body, nblocks, rows),
        grid=(2, nblocks),
        in_specs=[
            pl.BlockSpec((rows, c), lambda p, j: (jnp.where(p == 0, j, 0), 0)),
            pl.BlockSpec((1, 1, rows), lambda p, j: (j, 0, 0)),
            pl.BlockSpec((1, c), lambda p, j: (0, 0)),
            pl.BlockSpec((1, c), lambda p, j: (0, 0)),
        ],
        out_specs=pl.BlockSpec((rows, c),
                               lambda p, j: (jnp.where(p == 0, 0, j), 0)),
        out_shape=jax.ShapeDtypeStruct((n, c), jnp.float32),
        scratch_shapes=[
            pltpu.VMEM((n, c), jnp.float32),
            pltpu.VMEM((NSEG, c), jnp.float32),
            pltpu.VMEM((NSEG, c), jnp.float32),
            pltpu.VMEM((NSEG, c), jnp.float32),
        ],
        compiler_params=pltpu.CompilerParams(
            dimension_semantics=("arbitrary", "arbitrary")),
    )(data, bid3, weights, bias)
    return out
